# baseline (speedup 1.0000x reference)
import functools

import jax
import jax.numpy as jnp
from jax import lax
from jax.experimental import pallas as pl
from jax.experimental.pallas import tpu as pltpu
from jax.experimental.pallas import tpu_sc as plsc

_NUM_CORES = 2
_NUM_SUBCORES = 16
_NUM_WORKERS = _NUM_CORES * _NUM_SUBCORES
_LANES = 16
_GATHER_CHUNK = 128


@functools.lru_cache(maxsize=None)
def _build(batch, vocab, dim):
  assert batch % (_NUM_WORKERS * _LANES) == 0
  b_per_w = batch // _NUM_WORKERS
  n_chunks = b_per_w // _GATHER_CHUNK
  assert b_per_w % _GATHER_CHUNK == 0

  mesh = plsc.VectorSubcoreMesh(core_axis_name="c", subcore_axis_name="s")

  @functools.partial(
      pl.kernel,
      out_type=jax.ShapeDtypeStruct((batch,), jnp.float32),
      mesh=mesh,
      compiler_params=pltpu.CompilerParams(
          needs_layout_passes=False, use_tc_tiling_on_sc=False),
      scratch_types=[
          pltpu.VMEM((b_per_w,), jnp.int32),
          pltpu.VMEM((b_per_w,), jnp.int32),
          pltpu.VMEM((b_per_w, dim), jnp.float32),
          pltpu.VMEM((b_per_w, dim), jnp.float32),
          pltpu.VMEM((b_per_w,), jnp.float32),
          pltpu.SemaphoreType.DMA,
      ],
  )
  def scored(cidx_hbm, tidx_hbm, cemb_hbm, temb_hbm, out_hbm,
             cidx_v, tidx_v, crows_v, trows_v, out_v, sem):
    wid = lax.axis_index("s") * _NUM_CORES + lax.axis_index("c")
    base = wid * b_per_w

    pltpu.sync_copy(cidx_hbm.at[pl.ds(base, b_per_w)], cidx_v)
    pltpu.sync_copy(tidx_hbm.at[pl.ds(base, b_per_w)], tidx_v)

    copies = []
    for j in range(n_chunks):
      sl = pl.ds(j * _GATHER_CHUNK, _GATHER_CHUNK)
      copies.append(pltpu.async_copy(cemb_hbm.at[cidx_v.at[sl]], crows_v.at[sl], sem))
      copies.append(pltpu.async_copy(temb_hbm.at[tidx_v.at[sl]], trows_v.at[sl], sem))
    for c in copies:
      c.wait()

    lane = lax.iota(jnp.int32, 16)

    def group_body(g, carry):
      rows = jnp.full((_LANES,), g * _LANES, jnp.int32) + lane
      acc = [jnp.zeros((_LANES,), jnp.float32) for _ in range(4)]
      for d in range(dim):
        col = jnp.full((_LANES,), d, jnp.int32)
        cv = plsc.load_gather(crows_v, [rows, col])
        tv = plsc.load_gather(trows_v, [rows, col])
        acc[d % 4] = acc[d % 4] + cv * tv
      out_v[pl.ds(pl.multiple_of(g * _LANES, _LANES), _LANES)] = (
          (acc[0] + acc[1]) + (acc[2] + acc[3]))
      return carry

    lax.fori_loop(0, b_per_w // _LANES, group_body, 0)

    pltpu.sync_copy(out_v, out_hbm.at[pl.ds(base, b_per_w)])

  return scored


def kernel(center_word_idx, target_word_idx, input_embeddings, output_embeddings):
  batch = center_word_idx.shape[0]
  vocab, dim = input_embeddings.shape
  scored = _build(batch, vocab, dim)
  return scored(
      center_word_idx.astype(jnp.int32),
      target_word_idx.astype(jnp.int32),
      input_embeddings,
      output_embeddings,
  )
